# baseline (speedup 1.0000x reference)
import jax
import jax.numpy as jnp
from jax.experimental import pallas as pl
from jax.experimental.pallas import tpu as pltpu

_EPS = 1e-6
_H = 4
_W = 64
_M = 128
_BB = 256


_BSUB = 8


def _dh_block(hid_ref, mem_ref, wk_ref, bk_ref, wbx_ref, bbx_ref, out_ref,
              keys_ref, betas_ref):
    hid = hid_ref[...]
    keys_ref[...] = jnp.tanh(
        jnp.dot(hid, wk_ref[...], preferred_element_type=jnp.float32)
        + bk_ref[...]
    )
    betas_ref[...] = jax.nn.softplus(
        jnp.dot(hid, wbx_ref[...], preferred_element_type=jnp.float32)
        + bbx_ref[...]
    )

    for c in range(_BB // _BSUB):
        sl = slice(c * _BSUB, (c + 1) * _BSUB)
        memc = mem_ref[sl]
        v2 = jnp.sum(memc * memc, axis=-1) + _EPS
        keysc = keys_ref[sl]
        for h in range(_H):
            key_h = keysc[:, h * _W:(h + 1) * _W]
            u2 = jnp.sum(key_h * key_h, axis=-1, keepdims=True) + _EPS
            num = jnp.sum(memc * key_h[:, None, :], axis=-1)
            den = jnp.sqrt(u2 * v2) + _EPS
            s = (num / den) * betas_ref[sl, h * _M:(h + 1) * _M]
            mx = jnp.max(s, axis=-1, keepdims=True)
            e = jnp.exp(s - mx)
            out_ref[sl, h * _M:(h + 1) * _M] = (
                e / jnp.sum(e, axis=-1, keepdims=True))


def kernel(hidden_vb, memory_vb, W_key, b_key, W_beta, b_beta):
    B, D = hidden_vb.shape
    M, W = memory_vb.shape[1], memory_vb.shape[2]
    wbx = jnp.repeat(W_beta, _M, axis=1)
    bbx = jnp.repeat(b_beta, _M)[None, :]
    bk = b_key[None, :]

    grid = (B // _BB,)
    out = pl.pallas_call(
        _dh_block,
        grid=grid,
        in_specs=[
            pl.BlockSpec((_BB, D), lambda i: (i, 0)),
            pl.BlockSpec((_BB, M, W), lambda i: (i, 0, 0)),
            pl.BlockSpec((D, _H * _W), lambda i: (0, 0)),
            pl.BlockSpec((1, _H * _W), lambda i: (0, 0)),
            pl.BlockSpec((D, _H * _M), lambda i: (0, 0)),
            pl.BlockSpec((1, _H * _M), lambda i: (0, 0)),
        ],
        out_specs=pl.BlockSpec((_BB, _H * _M), lambda i: (i, 0)),
        out_shape=jax.ShapeDtypeStruct((B, _H * _M), jnp.float32),
        scratch_shapes=[
            pltpu.VMEM((_BB, _H * _W), jnp.float32),
            pltpu.VMEM((_BB, _H * _M), jnp.float32),
        ],
        compiler_params=pltpu.CompilerParams(
            dimension_semantics=("parallel",),
            vmem_limit_bytes=56 * 1024 * 1024,
        ),
    )(hidden_vb, memory_vb, W_key, bk, wbx, bbx)
    return out.reshape(B, _H, M)


# speedup vs baseline: 5.5445x; 5.5445x over previous
import jax
import jax.numpy as jnp
from jax.experimental import pallas as pl
from jax.experimental.pallas import tpu as pltpu

_EPS = 1e-6
_H = 4
_W = 64
_M = 128
_BB = 256
_BSUB = 16


def _dh_block(hid_ref, mem_ref, wk_ref, bk_ref, wbx_ref, bbx_ref, out_ref,
              keys_ref, betas_ref):
    hid = hid_ref[...]
    keys_ref[...] = jnp.tanh(
        jnp.dot(hid, wk_ref[...], preferred_element_type=jnp.float32)
        + bk_ref[...]
    )
    betas_ref[...] = jax.nn.softplus(
        jnp.dot(hid, wbx_ref[...], preferred_element_type=jnp.float32)
        + bbx_ref[...]
    )
    u2 = [
        jnp.sum(keys_ref[:, h * _W:(h + 1) * _W] ** 2, axis=-1,
                keepdims=True) + _EPS
        for h in range(_H)
    ]

    for c in range(_BB // _BSUB):
        sl = slice(c * _BSUB, (c + 1) * _BSUB)
        memT = jnp.swapaxes(mem_ref[sl], 1, 2)
        v2 = jnp.sum(memT * memT, axis=1) + _EPS
        for h in range(_H):
            kh = keys_ref[sl, h * _W:(h + 1) * _W]
            num = jax.lax.dot_general(
                kh.reshape(_BSUB, 1, _W), memT,
                (((2,), (1,)), ((0,), (0,))),
                preferred_element_type=jnp.float32,
            ).reshape(_BSUB, _M)
            den = jnp.sqrt(u2[h][sl] * v2) + _EPS
            s = (num / den) * betas_ref[sl, h * _M:(h + 1) * _M]
            mx = jnp.max(s, axis=-1, keepdims=True)
            e = jnp.exp(s - mx)
            out_ref[sl, h * _M:(h + 1) * _M] = (
                e / jnp.sum(e, axis=-1, keepdims=True))


def kernel(hidden_vb, memory_vb, W_key, b_key, W_beta, b_beta):
    B, D = hidden_vb.shape
    M, W = memory_vb.shape[1], memory_vb.shape[2]
    wbx = jnp.repeat(W_beta, _M, axis=1)
    bbx = jnp.repeat(b_beta, _M)[None, :]
    bk = b_key[None, :]

    grid = (B // _BB,)
    out = pl.pallas_call(
        _dh_block,
        grid=grid,
        in_specs=[
            pl.BlockSpec((_BB, D), lambda i: (i, 0)),
            pl.BlockSpec((_BB, M, W), lambda i: (i, 0, 0)),
            pl.BlockSpec((D, _H * _W), lambda i: (0, 0)),
            pl.BlockSpec((1, _H * _W), lambda i: (0, 0)),
            pl.BlockSpec((D, _H * _M), lambda i: (0, 0)),
            pl.BlockSpec((1, _H * _M), lambda i: (0, 0)),
        ],
        out_specs=pl.BlockSpec((_BB, _H * _M), lambda i: (i, 0)),
        out_shape=jax.ShapeDtypeStruct((B, _H * _M), jnp.float32),
        scratch_shapes=[
            pltpu.VMEM((_BB, _H * _W), jnp.float32),
            pltpu.VMEM((_BB, _H * _M), jnp.float32),
        ],
        compiler_params=pltpu.CompilerParams(
            dimension_semantics=("parallel",),
            vmem_limit_bytes=56 * 1024 * 1024,
        ),
    )(hidden_vb, memory_vb, W_key, bk, wbx, bbx)
    return out.reshape(B, _H, M)


# speedup vs baseline: 6.1763x; 1.1139x over previous
import jax
import jax.numpy as jnp
from jax.experimental import pallas as pl
from jax.experimental.pallas import tpu as pltpu

_EPS = 1e-6
_H = 4
_W = 64
_M = 128
_BB = 256
_BSUB = 16


def _dh_block(hid_ref, mem_ref, wk_ref, bk_ref, wbx_ref, bbx_ref, out_ref,
              keys_ref, betas_ref):
    hid = hid_ref[...]
    keys_ref[...] = jnp.tanh(
        jnp.dot(hid, wk_ref[...], preferred_element_type=jnp.float32)
        + bk_ref[...]
    )
    betas_ref[...] = jax.nn.softplus(
        jnp.dot(hid, wbx_ref[...], preferred_element_type=jnp.float32)
        + bbx_ref[...]
    )
    u2 = [
        jnp.sum(keys_ref[:, h * _W:(h + 1) * _W] ** 2, axis=-1,
                keepdims=True) + _EPS
        for h in range(_H)
    ]

    for c in range(_BB // _BSUB):
        sl = slice(c * _BSUB, (c + 1) * _BSUB)
        memT = jnp.swapaxes(mem_ref[sl], 1, 2)
        v2 = jnp.sum(memT * memT, axis=1) + _EPS
        khall = jnp.concatenate(
            [keys_ref[sl, h * _W:(h + 1) * _W][:, None, :]
             for h in range(_H)], axis=1)
        numall = jax.lax.dot_general(
            khall, memT, (((2,), (1,)), ((0,), (0,))),
            preferred_element_type=jnp.float32,
        )
        for h in range(_H):
            num = numall[:, h, :]
            den = jnp.sqrt(u2[h][sl] * v2) + _EPS
            s = (num / den) * betas_ref[sl, h * _M:(h + 1) * _M]
            mx = jnp.max(s, axis=-1, keepdims=True)
            e = jnp.exp(s - mx)
            out_ref[sl, h * _M:(h + 1) * _M] = (
                e / jnp.sum(e, axis=-1, keepdims=True))


def kernel(hidden_vb, memory_vb, W_key, b_key, W_beta, b_beta):
    B, D = hidden_vb.shape
    M, W = memory_vb.shape[1], memory_vb.shape[2]
    wbx = jnp.repeat(W_beta, _M, axis=1)
    bbx = jnp.repeat(b_beta, _M)[None, :]
    bk = b_key[None, :]

    grid = (B // _BB,)
    out = pl.pallas_call(
        _dh_block,
        grid=grid,
        in_specs=[
            pl.BlockSpec((_BB, D), lambda i: (i, 0)),
            pl.BlockSpec((_BB, M, W), lambda i: (i, 0, 0)),
            pl.BlockSpec((D, _H * _W), lambda i: (0, 0)),
            pl.BlockSpec((1, _H * _W), lambda i: (0, 0)),
            pl.BlockSpec((D, _H * _M), lambda i: (0, 0)),
            pl.BlockSpec((1, _H * _M), lambda i: (0, 0)),
        ],
        out_specs=pl.BlockSpec((_BB, _H * _M), lambda i: (i, 0)),
        out_shape=jax.ShapeDtypeStruct((B, _H * _M), jnp.float32),
        scratch_shapes=[
            pltpu.VMEM((_BB, _H * _W), jnp.float32),
            pltpu.VMEM((_BB, _H * _M), jnp.float32),
        ],
        compiler_params=pltpu.CompilerParams(
            dimension_semantics=("parallel",),
            vmem_limit_bytes=56 * 1024 * 1024,
        ),
    )(hidden_vb, memory_vb, W_key, bk, wbx, bbx)
    return out.reshape(B, _H, M)
